# baseline (speedup 1.0000x reference)
import functools

import jax
import jax.numpy as jnp
from jax import lax
from jax.experimental import pallas as pl
from jax.experimental.pallas import tpu as pltpu
from jax.experimental.pallas import tpu_sc as plsc

S, B, H = 2048, 1, 1024
E, K, F = 64, 2, 1408
CAP = 128
T = S * B
NSLOT = E * CAP
NROWS = NSLOT + CAP
NEOUT = NSLOT + CAP
NW = 32
TPB = T // NW
CH = 16
LANES = 128


def _router_body(x_ref, rw_ref, s0_ref, s1_ref, c0_ref, c1_ref,
                 w0_ref, w1_ref, xc_ref):
    x = x_ref[...]
    rw = rw_ref[...]
    logits = lax.dot_general(x, rw, (((1,), (1,)), ((), ())),
                             preferred_element_type=jnp.float32)
    lane = lax.broadcasted_iota(jnp.int32, (T, E), 1)
    m0 = jnp.max(logits, axis=1, keepdims=True)
    i0 = jnp.min(jnp.where(logits == m0, lane, E), axis=1, keepdims=True)
    h0 = lane == i0
    l2 = jnp.where(h0, -1e30, logits)
    m1 = jnp.max(l2, axis=1, keepdims=True)
    i1 = jnp.min(jnp.where(l2 == m1, lane, E), axis=1, keepdims=True)
    h1 = lane == i1
    d = jnp.exp(m1 - m0)
    v0 = 1.0 / (1.0 + d)
    v1 = 1.0 - v0
    m = jnp.where(h0 | h1, 1.0, 0.0)
    c = m
    sh = 1
    while sh < T:
        c = c + jnp.concatenate(
            [jnp.zeros((sh, E), jnp.float32), c[:T - sh]], axis=0)
        sh *= 2
    posm = c - 1.0
    pos0 = jnp.sum(jnp.where(h0, posm, 0.0), axis=1,
                   keepdims=True).astype(jnp.int32)
    pos1 = jnp.sum(jnp.where(h1, posm, 0.0), axis=1,
                   keepdims=True).astype(jnp.int32)
    drop0 = pos0 >= CAP
    drop1 = pos1 >= CAP
    s0 = i0 * CAP + pos0
    s1 = i1 * CAP + pos1
    slot0 = jnp.where(drop0, NSLOT, s0)
    slot1 = jnp.where(drop1, NSLOT + 1, s1)
    cslot0 = jnp.where(drop0, NSLOT, s0)
    cslot1 = jnp.where(drop1, NSLOT, s1)
    w0 = jnp.where(drop0, 0.0, v0)
    w1 = jnp.where(drop1, 0.0, v1)
    s0_ref[...] = slot0.reshape(T // LANES, LANES)
    s1_ref[...] = slot1.reshape(T // LANES, LANES)
    c0_ref[...] = cslot0.reshape(T // LANES, LANES)
    c1_ref[...] = cslot1.reshape(T // LANES, LANES)
    w0_ref[...] = jnp.broadcast_to(w0, (T, LANES))
    w1_ref[...] = jnp.broadcast_to(w1, (T, LANES))
    xc_ref[...] = x


def _router(x, router_w):
    i32b = jax.ShapeDtypeStruct((T // LANES, LANES), jnp.int32)
    f32b = jax.ShapeDtypeStruct((T, LANES), jnp.float32)
    return pl.pallas_call(
        _router_body,
        out_shape=(i32b, i32b, i32b, i32b, f32b, f32b,
                   jax.ShapeDtypeStruct((T, H), jnp.float32)),
    )(x, router_w)


def _dispatch(x, slot0b, slot1b):
    mesh = plsc.VectorSubcoreMesh(core_axis_name="c", subcore_axis_name="s")

    @functools.partial(
        pl.kernel,
        out_type=jax.ShapeDtypeStruct((NROWS, H), jnp.float32),
        mesh=mesh,
        scratch_types=[
            pltpu.VMEM((TPB, H), jnp.float32),
            pltpu.VMEM((TPB,), jnp.int32),
            pltpu.VMEM((TPB,), jnp.int32),
            pltpu.SemaphoreType.DMA,
        ],
    )
    def k(x_hbm, s0_hbm, s1_hbm, out_hbm,
          rows_v, i0_v, i1_v, sem):
        wid = lax.axis_index("s") * 2 + lax.axis_index("c")
        base = wid * TPB
        pltpu.sync_copy(x_hbm.at[pl.ds(base, TPB)], rows_v)
        pltpu.sync_copy(s0_hbm.at[pl.ds(base, TPB)], i0_v)
        pltpu.sync_copy(s1_hbm.at[pl.ds(base, TPB)], i1_v)
        a = pltpu.async_copy(rows_v, out_hbm.at[i0_v], sem)
        b = pltpu.async_copy(rows_v, out_hbm.at[i1_v], sem)
        a.wait()
        b.wait()

    return k(x, slot0b, slot1b)


def _experts_body(in_ref, wg_ref, wu_ref, wd_ref, out_ref):
    xin = in_ref[...].astype(jnp.bfloat16)
    wg = wg_ref[0].astype(jnp.bfloat16)
    wu = wu_ref[0].astype(jnp.bfloat16)
    g = jnp.dot(xin, wg, preferred_element_type=jnp.float32)
    u = jnp.dot(xin, wu, preferred_element_type=jnp.float32)
    h = g * (1.0 / (1.0 + jnp.exp(-g))) * u
    o = jnp.dot(h.astype(jnp.bfloat16), wd_ref[0].astype(jnp.bfloat16),
                preferred_element_type=jnp.float32)
    scale = jnp.where(pl.program_id(0) < E, 1.0, 0.0)
    out_ref[...] = o * scale


def _experts(expert_in, w_gate, w_up, w_down):
    last = E - 1
    return pl.pallas_call(
        _experts_body,
        grid=(E + 1,),
        in_specs=[
            pl.BlockSpec((CAP, H), lambda e: (jnp.minimum(e, last), 0)),
            pl.BlockSpec((1, H, F), lambda e: (jnp.minimum(e, last), 0, 0)),
            pl.BlockSpec((1, H, F), lambda e: (jnp.minimum(e, last), 0, 0)),
            pl.BlockSpec((1, F, H), lambda e: (jnp.minimum(e, last), 0, 0)),
        ],
        out_specs=pl.BlockSpec((CAP, H), lambda e: (e, 0)),
        out_shape=jax.ShapeDtypeStruct((NEOUT, H), jnp.float32),
        compiler_params=pltpu.CompilerParams(
            dimension_semantics=("arbitrary",)),
    )(expert_in, w_gate, w_up, w_down)


def _combine(eout, cslot0b, cslot1b, w0b, w1b):
    mesh = plsc.VectorSubcoreMesh(core_axis_name="c", subcore_axis_name="s")

    @functools.partial(
        pl.kernel,
        out_type=jax.ShapeDtypeStruct((T, H), jnp.float32),
        mesh=mesh,
        scratch_types=[
            pltpu.VMEM((TPB,), jnp.int32),
            pltpu.VMEM((TPB,), jnp.int32),
            pltpu.VMEM((TPB, LANES), jnp.float32),
            pltpu.VMEM((TPB, LANES), jnp.float32),
            pltpu.VMEM((2, CH, H), jnp.float32),
            pltpu.VMEM((2, CH, H), jnp.float32),
            pltpu.SemaphoreType.DMA,
        ],
    )
    def k(eout_hbm, c0_hbm, c1_hbm, w0_hbm, w1_hbm, out_hbm,
          i0_v, i1_v, w0_v, w1_v, r0_v, r1_v, sem):
        wid = lax.axis_index("s") * 2 + lax.axis_index("c")
        base = wid * TPB
        pltpu.sync_copy(c0_hbm.at[pl.ds(base, TPB)], i0_v)
        pltpu.sync_copy(c1_hbm.at[pl.ds(base, TPB)], i1_v)
        pltpu.sync_copy(w0_hbm.at[pl.ds(base, TPB)], w0_v)
        pltpu.sync_copy(w1_hbm.at[pl.ds(base, TPB)], w1_v)

        nch = TPB // CH

        def gather(c, buf):
            a = pltpu.async_copy(
                eout_hbm.at[i0_v.at[pl.ds(c * CH, CH)]], r0_v.at[buf], sem)
            b = pltpu.async_copy(
                eout_hbm.at[i1_v.at[pl.ds(c * CH, CH)]], r1_v.at[buf], sem)
            return a, b

        pend = gather(0, 0)
        for c in range(nch):
            buf = c % 2
            pend[0].wait()
            pend[1].wait()
            if c + 1 < nch:
                pend = gather(c + 1, 1 - buf)

            def tok(i, _, c=c, buf=buf):
                w0 = w0_v[c * CH + i, pl.ds(0, 16)]
                w1 = w1_v[c * CH + i, pl.ds(0, 16)]
                for j in range(H // 16):
                    r0 = r0_v[buf, i, pl.ds(j * 16, 16)]
                    r1 = r1_v[buf, i, pl.ds(j * 16, 16)]
                    r0_v[buf, i, pl.ds(j * 16, 16)] = w0 * r0 + w1 * r1
                return 0

            lax.fori_loop(0, CH, tok, 0)
            pltpu.sync_copy(r0_v.at[buf],
                            out_hbm.at[pl.ds(base + c * CH, CH)])

    return k(eout, cslot0b, cslot1b, w0b, w1b)


def _finalize_body(x_ref, o_ref):
    o_ref[...] = x_ref[...]


def _finalize(x):
    return pl.pallas_call(
        _finalize_body,
        out_shape=jax.ShapeDtypeStruct((T, H), jnp.float32))(x)


def kernel(hidden_states, router_w, w_gate, w_up, w_down):
    x = hidden_states.reshape(T, H)
    (slot0b, slot1b, cslot0b, cslot1b, w0b, w1b,
     x_copy) = _router(x, router_w)
    expert_in = _dispatch(x_copy, slot0b.reshape(-1), slot1b.reshape(-1))
    eout = _experts(expert_in, w_gate, w_up, w_down)
    out = _combine(eout, cslot0b.reshape(-1), cslot1b.reshape(-1),
                   w0b, w1b)
    return _finalize(out).reshape(S, B, H)


# speedup vs baseline: 1.0178x; 1.0178x over previous
import functools

import jax
import jax.numpy as jnp
from jax import lax
from jax.experimental import pallas as pl
from jax.experimental.pallas import tpu as pltpu
from jax.experimental.pallas import tpu_sc as plsc

S, B, H = 2048, 1, 1024
E, K, F = 64, 2, 1408
CAP = 128
T = S * B
NSLOT = E * CAP
NROWS = NSLOT + CAP
NEOUT = NSLOT + CAP
NW = 32
TPB = T // NW
CH = 16
LANES = 128


def _router_body(x_ref, rw_ref, s0_ref, s1_ref, c0_ref, c1_ref,
                 w0_ref, w1_ref):
    x = x_ref[...]
    rw = rw_ref[...]
    logits = lax.dot_general(x, rw, (((1,), (1,)), ((), ())),
                             preferred_element_type=jnp.float32)
    lane = lax.broadcasted_iota(jnp.int32, (T, E), 1)
    m0 = jnp.max(logits, axis=1, keepdims=True)
    i0 = jnp.min(jnp.where(logits == m0, lane, E), axis=1, keepdims=True)
    h0 = lane == i0
    l2 = jnp.where(h0, -1e30, logits)
    m1 = jnp.max(l2, axis=1, keepdims=True)
    i1 = jnp.min(jnp.where(l2 == m1, lane, E), axis=1, keepdims=True)
    h1 = lane == i1
    d = jnp.exp(m1 - m0)
    v0 = 1.0 / (1.0 + d)
    v1 = 1.0 - v0
    m = jnp.where(h0 | h1, 1.0, 0.0)
    c = m
    sh = 1
    while sh < T:
        c = c + jnp.concatenate(
            [jnp.zeros((sh, E), jnp.float32), c[:T - sh]], axis=0)
        sh *= 2
    posm = c - 1.0
    pos0 = jnp.sum(jnp.where(h0, posm, 0.0), axis=1,
                   keepdims=True).astype(jnp.int32)
    pos1 = jnp.sum(jnp.where(h1, posm, 0.0), axis=1,
                   keepdims=True).astype(jnp.int32)
    drop0 = pos0 >= CAP
    drop1 = pos1 >= CAP
    s0 = i0 * CAP + pos0
    s1 = i1 * CAP + pos1
    slot0 = jnp.where(drop0, NSLOT, s0)
    slot1 = jnp.where(drop1, NSLOT + 1, s1)
    cslot0 = jnp.where(drop0, NSLOT, s0)
    cslot1 = jnp.where(drop1, NSLOT, s1)
    w0 = jnp.where(drop0, 0.0, v0)
    w1 = jnp.where(drop1, 0.0, v1)
    s0_ref[...] = slot0.reshape(T // LANES, LANES)
    s1_ref[...] = slot1.reshape(T // LANES, LANES)
    c0_ref[...] = cslot0.reshape(T // LANES, LANES)
    c1_ref[...] = cslot1.reshape(T // LANES, LANES)
    w0_ref[...] = jnp.broadcast_to(w0, (T, LANES))
    w1_ref[...] = jnp.broadcast_to(w1, (T, LANES))


def _router(x, router_w):
    i32b = jax.ShapeDtypeStruct((T // LANES, LANES), jnp.int32)
    f32b = jax.ShapeDtypeStruct((T, LANES), jnp.float32)
    return pl.pallas_call(
        _router_body,
        out_shape=(i32b, i32b, i32b, i32b, f32b, f32b),
    )(x, router_w)


def _dispatch(x, slot0b, slot1b):
    mesh = plsc.VectorSubcoreMesh(core_axis_name="c", subcore_axis_name="s")

    @functools.partial(
        pl.kernel,
        out_type=jax.ShapeDtypeStruct((NROWS, H), jnp.float32),
        mesh=mesh,
        scratch_types=[
            pltpu.VMEM((TPB, H), jnp.float32),
            pltpu.VMEM((TPB,), jnp.int32),
            pltpu.VMEM((TPB,), jnp.int32),
            pltpu.SemaphoreType.DMA,
        ],
    )
    def k(x_hbm, s0_hbm, s1_hbm, out_hbm,
          rows_v, i0_v, i1_v, sem):
        wid = lax.axis_index("s") * 2 + lax.axis_index("c")
        base = wid * TPB
        pltpu.sync_copy(x_hbm.at[pl.ds(base, TPB)], rows_v)
        pltpu.sync_copy(s0_hbm.at[pl.ds(base, TPB)], i0_v)
        pltpu.sync_copy(s1_hbm.at[pl.ds(base, TPB)], i1_v)
        a = pltpu.async_copy(rows_v, out_hbm.at[i0_v], sem)
        b = pltpu.async_copy(rows_v, out_hbm.at[i1_v], sem)
        a.wait()
        b.wait()

    return k(x, slot0b, slot1b)


def _experts_body(in_ref, wg_ref, wu_ref, wd_ref, out_ref):
    xin = in_ref[...].astype(jnp.bfloat16)
    wg = wg_ref[0].astype(jnp.bfloat16)
    wu = wu_ref[0].astype(jnp.bfloat16)
    g = jnp.dot(xin, wg, preferred_element_type=jnp.float32)
    u = jnp.dot(xin, wu, preferred_element_type=jnp.float32)
    h = g * (1.0 / (1.0 + jnp.exp(-g))) * u
    o = jnp.dot(h.astype(jnp.bfloat16), wd_ref[0].astype(jnp.bfloat16),
                preferred_element_type=jnp.float32)
    scale = jnp.where(pl.program_id(0) < E, 1.0, 0.0)
    out_ref[...] = o * scale


def _experts(expert_in, w_gate, w_up, w_down):
    last = E - 1
    return pl.pallas_call(
        _experts_body,
        grid=(E + 1,),
        in_specs=[
            pl.BlockSpec((CAP, H), lambda e: (jnp.minimum(e, last), 0)),
            pl.BlockSpec((1, H, F), lambda e: (jnp.minimum(e, last), 0, 0)),
            pl.BlockSpec((1, H, F), lambda e: (jnp.minimum(e, last), 0, 0)),
            pl.BlockSpec((1, F, H), lambda e: (jnp.minimum(e, last), 0, 0)),
        ],
        out_specs=pl.BlockSpec((CAP, H), lambda e: (e, 0)),
        out_shape=jax.ShapeDtypeStruct((NEOUT, H), jnp.float32),
        compiler_params=pltpu.CompilerParams(
            dimension_semantics=("arbitrary",)),
    )(expert_in, w_gate, w_up, w_down)


def _combine(eout, cslot0b, cslot1b, w0b, w1b):
    mesh = plsc.VectorSubcoreMesh(core_axis_name="c", subcore_axis_name="s")

    @functools.partial(
        pl.kernel,
        out_type=jax.ShapeDtypeStruct((T, H), jnp.float32),
        mesh=mesh,
        scratch_types=[
            pltpu.VMEM((TPB,), jnp.int32),
            pltpu.VMEM((TPB,), jnp.int32),
            pltpu.VMEM((TPB, LANES), jnp.float32),
            pltpu.VMEM((TPB, LANES), jnp.float32),
            pltpu.VMEM((2, CH, H), jnp.float32),
            pltpu.VMEM((2, CH, H), jnp.float32),
            pltpu.SemaphoreType.DMA,
        ],
    )
    def k(eout_hbm, c0_hbm, c1_hbm, w0_hbm, w1_hbm, out_hbm,
          i0_v, i1_v, w0_v, w1_v, r0_v, r1_v, sem):
        wid = lax.axis_index("s") * 2 + lax.axis_index("c")
        base = wid * TPB
        pltpu.sync_copy(c0_hbm.at[pl.ds(base, TPB)], i0_v)
        pltpu.sync_copy(c1_hbm.at[pl.ds(base, TPB)], i1_v)
        pltpu.sync_copy(w0_hbm.at[pl.ds(base, TPB)], w0_v)
        pltpu.sync_copy(w1_hbm.at[pl.ds(base, TPB)], w1_v)

        nch = TPB // CH

        def gather(c, buf):
            a = pltpu.async_copy(
                eout_hbm.at[i0_v.at[pl.ds(c * CH, CH)]], r0_v.at[buf], sem)
            b = pltpu.async_copy(
                eout_hbm.at[i1_v.at[pl.ds(c * CH, CH)]], r1_v.at[buf], sem)
            return a, b

        pend = gather(0, 0)
        for c in range(nch):
            buf = c % 2
            pend[0].wait()
            pend[1].wait()
            if c + 1 < nch:
                pend = gather(c + 1, 1 - buf)

            def tok(i, _, c=c, buf=buf):
                w0 = w0_v[c * CH + i, pl.ds(0, 16)]
                w1 = w1_v[c * CH + i, pl.ds(0, 16)]
                for j in range(H // 16):
                    r0 = r0_v[buf, i, pl.ds(j * 16, 16)]
                    r1 = r1_v[buf, i, pl.ds(j * 16, 16)]
                    r0_v[buf, i, pl.ds(j * 16, 16)] = w0 * r0 + w1 * r1
                return 0

            lax.fori_loop(0, CH, tok, 0)
            pltpu.sync_copy(r0_v.at[buf],
                            out_hbm.at[pl.ds(base + c * CH, CH)])

    return k(eout, cslot0b, cslot1b, w0b, w1b)


def kernel(hidden_states, router_w, w_gate, w_up, w_down):
    x = hidden_states.reshape(T, H)
    slot0b, slot1b, cslot0b, cslot1b, w0b, w1b = _router(x, router_w)
    expert_in = _dispatch(x, slot0b.reshape(-1), slot1b.reshape(-1))
    eout = _experts(expert_in, w_gate, w_up, w_down)
    out = _combine(eout, cslot0b.reshape(-1), cslot1b.reshape(-1),
                   w0b, w1b)
    return out.reshape(S, B, H)


# speedup vs baseline: 1.0185x; 1.0008x over previous
import functools

import jax
import jax.numpy as jnp
from jax import lax
from jax.experimental import pallas as pl
from jax.experimental.pallas import tpu as pltpu
from jax.experimental.pallas import tpu_sc as plsc

S, B, H = 2048, 1, 1024
E, K, F = 64, 2, 1408
CAP = 128
T = S * B
NSLOT = E * CAP
NROWS = NSLOT + CAP
NEOUT = NSLOT + CAP
NW = 32
TPB = T // NW
CH = 16
LANES = 128


def _router_body(x_ref, rw_ref, s0_ref, s1_ref, c0_ref, c1_ref,
                 w0_ref, w1_ref):
    x = x_ref[...]
    rw = rw_ref[...]
    logits = lax.dot_general(x, rw, (((1,), (1,)), ((), ())),
                             preferred_element_type=jnp.float32)
    lane = lax.broadcasted_iota(jnp.int32, (T, E), 1)
    m0 = jnp.max(logits, axis=1, keepdims=True)
    i0 = jnp.min(jnp.where(logits == m0, lane, E), axis=1, keepdims=True)
    h0 = lane == i0
    l2 = jnp.where(h0, -1e30, logits)
    m1 = jnp.max(l2, axis=1, keepdims=True)
    i1 = jnp.min(jnp.where(l2 == m1, lane, E), axis=1, keepdims=True)
    h1 = lane == i1
    d = jnp.exp(m1 - m0)
    v0 = 1.0 / (1.0 + d)
    v1 = 1.0 - v0
    m = jnp.where(h0 | h1, 1.0, 0.0)
    c = m
    sh = 1
    while sh < T:
        c = c + jnp.concatenate(
            [jnp.zeros((sh, E), jnp.float32), c[:T - sh]], axis=0)
        sh *= 2
    posm = c - 1.0
    pos0 = jnp.sum(jnp.where(h0, posm, 0.0), axis=1,
                   keepdims=True).astype(jnp.int32)
    pos1 = jnp.sum(jnp.where(h1, posm, 0.0), axis=1,
                   keepdims=True).astype(jnp.int32)
    drop0 = pos0 >= CAP
    drop1 = pos1 >= CAP
    s0 = i0 * CAP + pos0
    s1 = i1 * CAP + pos1
    slot0 = jnp.where(drop0, NSLOT, s0)
    slot1 = jnp.where(drop1, NSLOT + 1, s1)
    cslot0 = jnp.where(drop0, NSLOT, s0)
    cslot1 = jnp.where(drop1, NSLOT, s1)
    w0 = jnp.where(drop0, 0.0, v0)
    w1 = jnp.where(drop1, 0.0, v1)
    s0_ref[...] = slot0.reshape(T // LANES, LANES)
    s1_ref[...] = slot1.reshape(T // LANES, LANES)
    c0_ref[...] = cslot0.reshape(T // LANES, LANES)
    c1_ref[...] = cslot1.reshape(T // LANES, LANES)
    w0_ref[...] = jnp.broadcast_to(w0, (T, LANES))
    w1_ref[...] = jnp.broadcast_to(w1, (T, LANES))


def _router(x, router_w):
    i32b = jax.ShapeDtypeStruct((T // LANES, LANES), jnp.int32)
    f32b = jax.ShapeDtypeStruct((T, LANES), jnp.float32)
    return pl.pallas_call(
        _router_body,
        out_shape=(i32b, i32b, i32b, i32b, f32b, f32b),
    )(x, router_w)


def _dispatch(x, slot0b, slot1b):
    mesh = plsc.VectorSubcoreMesh(core_axis_name="c", subcore_axis_name="s")

    @functools.partial(
        pl.kernel,
        out_type=jax.ShapeDtypeStruct((NROWS, H), jnp.float32),
        mesh=mesh,
        scratch_types=[
            pltpu.VMEM((TPB, H), jnp.float32),
            pltpu.VMEM((TPB,), jnp.int32),
            pltpu.VMEM((TPB,), jnp.int32),
            pltpu.SemaphoreType.DMA,
        ],
    )
    def k(x_hbm, s0_hbm, s1_hbm, out_hbm,
          rows_v, i0_v, i1_v, sem):
        wid = lax.axis_index("s") * 2 + lax.axis_index("c")
        base = wid * TPB
        pltpu.sync_copy(x_hbm.at[pl.ds(base, TPB)], rows_v)
        pltpu.sync_copy(s0_hbm.at[pl.ds(base, TPB)], i0_v)
        pltpu.sync_copy(s1_hbm.at[pl.ds(base, TPB)], i1_v)
        a = pltpu.async_copy(rows_v, out_hbm.at[i0_v], sem)
        b = pltpu.async_copy(rows_v, out_hbm.at[i1_v], sem)
        a.wait()
        b.wait()

    return k(x, slot0b, slot1b)


def _experts_body(in_ref, wg_ref, wu_ref, wd_ref, out_ref):
    xin = in_ref[...].astype(jnp.bfloat16)
    wg = wg_ref[0].astype(jnp.bfloat16)
    wu = wu_ref[0].astype(jnp.bfloat16)
    g = jnp.dot(xin, wg, preferred_element_type=jnp.float32)
    u = jnp.dot(xin, wu, preferred_element_type=jnp.float32)
    h = g * (1.0 / (1.0 + jnp.exp(-g))) * u
    o = jnp.dot(h.astype(jnp.bfloat16), wd_ref[0].astype(jnp.bfloat16),
                preferred_element_type=jnp.float32)
    scale = jnp.where(pl.program_id(0) < E, 1.0, 0.0)
    out_ref[...] = o * scale


def _experts(expert_in, w_gate, w_up, w_down):
    last = E - 1
    return pl.pallas_call(
        _experts_body,
        grid=(E + 1,),
        in_specs=[
            pl.BlockSpec((CAP, H), lambda e: (jnp.minimum(e, last), 0)),
            pl.BlockSpec((1, H, F), lambda e: (jnp.minimum(e, last), 0, 0)),
            pl.BlockSpec((1, H, F), lambda e: (jnp.minimum(e, last), 0, 0)),
            pl.BlockSpec((1, F, H), lambda e: (jnp.minimum(e, last), 0, 0)),
        ],
        out_specs=pl.BlockSpec((CAP, H), lambda e: (e, 0)),
        out_shape=jax.ShapeDtypeStruct((NEOUT, H), jnp.float32),
        compiler_params=pltpu.CompilerParams(
            dimension_semantics=("arbitrary",)),
    )(expert_in, w_gate, w_up, w_down)


def _combine(eout, cslot0b, cslot1b, w0b, w1b):
    mesh = plsc.VectorSubcoreMesh(core_axis_name="c", subcore_axis_name="s")

    @functools.partial(
        pl.kernel,
        out_type=jax.ShapeDtypeStruct((T, H), jnp.float32),
        mesh=mesh,
        scratch_types=[
            pltpu.VMEM((TPB,), jnp.int32),
            pltpu.VMEM((TPB,), jnp.int32),
            pltpu.VMEM((TPB, LANES), jnp.float32),
            pltpu.VMEM((TPB, LANES), jnp.float32),
            pltpu.VMEM((2, CH, H), jnp.float32),
            pltpu.VMEM((2, CH, H), jnp.float32),
            pltpu.SemaphoreType.DMA,
            pltpu.SemaphoreType.DMA,
        ],
    )
    def k(eout_hbm, c0_hbm, c1_hbm, w0_hbm, w1_hbm, out_hbm,
          i0_v, i1_v, w0_v, w1_v, r0_v, r1_v, sem, osem):
        wid = lax.axis_index("s") * 2 + lax.axis_index("c")
        base = wid * TPB
        pltpu.sync_copy(c0_hbm.at[pl.ds(base, TPB)], i0_v)
        pltpu.sync_copy(c1_hbm.at[pl.ds(base, TPB)], i1_v)
        pltpu.sync_copy(w0_hbm.at[pl.ds(base, TPB)], w0_v)
        pltpu.sync_copy(w1_hbm.at[pl.ds(base, TPB)], w1_v)

        nch = TPB // CH

        def gather(c, buf):
            a = pltpu.async_copy(
                eout_hbm.at[i0_v.at[pl.ds(c * CH, CH)]], r0_v.at[buf], sem)
            b = pltpu.async_copy(
                eout_hbm.at[i1_v.at[pl.ds(c * CH, CH)]], r1_v.at[buf], sem)
            return a, b

        pend = gather(0, 0)
        outp = [None] * nch
        for c in range(nch):
            buf = c % 2
            pend[0].wait()
            pend[1].wait()
            if c + 1 < nch:
                if c >= 1:
                    outp[c - 1].wait()
                pend = gather(c + 1, 1 - buf)

            def tok(i, _, c=c, buf=buf):
                w0 = w0_v[c * CH + i, pl.ds(0, 16)]
                w1 = w1_v[c * CH + i, pl.ds(0, 16)]
                for j in range(H // 16):
                    r0 = r0_v[buf, i, pl.ds(j * 16, 16)]
                    r1 = r1_v[buf, i, pl.ds(j * 16, 16)]
                    r0_v[buf, i, pl.ds(j * 16, 16)] = w0 * r0 + w1 * r1
                return 0

            lax.fori_loop(0, CH, tok, 0)
            outp[c] = pltpu.async_copy(
                r0_v.at[buf], out_hbm.at[pl.ds(base + c * CH, CH)], osem)
        outp[nch - 2].wait()
        outp[nch - 1].wait()

    return k(eout, cslot0b, cslot1b, w0b, w1b)


def kernel(hidden_states, router_w, w_gate, w_up, w_down):
    x = hidden_states.reshape(T, H)
    slot0b, slot1b, cslot0b, cslot1b, w0b, w1b = _router(x, router_w)
    expert_in = _dispatch(x, slot0b.reshape(-1), slot1b.reshape(-1))
    eout = _experts(expert_in, w_gate, w_up, w_down)
    out = _combine(eout, cslot0b.reshape(-1), cslot1b.reshape(-1),
                   w0b, w1b)
    return out.reshape(S, B, H)
